# baseline (speedup 1.0000x reference)
import functools

import jax
import jax.numpy as jnp
from jax import lax
from jax.experimental import pallas as pl
from jax.experimental.pallas import tpu as pltpu
from jax.experimental.pallas import tpu_sc as plsc

_B = 4096
_K = 20
_D = 64
_ITEM_N = 1000
_MARGIN = 0.5
_LAMBDA_F = 0.5
_LAMBDA_C = 1.0

_info = plsc.get_sparse_core_info()
_NC, _NS = _info.num_cores, _info.num_subcores
_NW = _NC * _NS
_BPW = _B // _NW


def _sc_gather(user_idx, item_idx, negk, emb_user, emb_item):
    mesh = plsc.VectorSubcoreMesh(core_axis_name="c", subcore_axis_name="s")

    @functools.partial(
        pl.kernel,
        out_type=[
            jax.ShapeDtypeStruct((_B, 128), jnp.float32),
            jax.ShapeDtypeStruct((_B, 128), jnp.float32),
            jax.ShapeDtypeStruct((_B * _K, 128), jnp.float32),
        ],
        mesh=mesh,
        compiler_params=pltpu.CompilerParams(
            use_tc_tiling_on_sc=False, needs_layout_passes=False),
        scratch_types=[
            pltpu.VMEM((_BPW,), jnp.int32),
            pltpu.VMEM((_BPW,), jnp.int32),
            pltpu.VMEM((_K * _BPW,), jnp.int32),
            pltpu.VMEM((_K * _BPW,), jnp.int32),
            pltpu.VMEM((_BPW, _D), jnp.float32),
            pltpu.VMEM((_BPW, _D), jnp.float32),
            pltpu.VMEM((2, _BPW, _D), jnp.float32),
            pltpu.SemaphoreType.DMA,
            pltpu.SemaphoreType.DMA,
            pltpu.SemaphoreType.DMA,
        ],
    )
    def sc_kern(uidx_hbm, iidx_hbm, nidx_hbm, eu_hbm, ei_hbm,
                ui_out, vj_out, vk_out,
                uidx_v, iidx_v, nraw_v, kidx_v, ui_v, vj_v, vk_v,
                sem_a, sem_b, sem_c):
        wid = lax.axis_index("s") * _NC + lax.axis_index("c")
        base = wid * _BPW

        cps = [
            pltpu.async_copy(uidx_hbm.at[pl.ds(base, _BPW)], uidx_v, sem_a),
            pltpu.async_copy(iidx_hbm.at[pl.ds(base, _BPW)], iidx_v, sem_a),
            pltpu.async_copy(
                nidx_hbm.at[pl.ds(base * _K, _BPW * _K)], nraw_v, sem_a),
        ]
        for c in cps:
            c.wait()

        g1 = pltpu.async_copy(eu_hbm.at[uidx_v], ui_v, sem_b)
        g2 = pltpu.async_copy(ei_hbm.at[iidx_v], vj_v, sem_b)

        lane = jnp.arange(16, dtype=jnp.int32) * _K
        for kk in range(_K):
            for c in range(_BPW // 16):
                vals = plsc.load_gather(nraw_v, [lane + (c * 16 * _K + kk)])
                kidx_v[pl.ds(kk * _BPW + c * 16, 16)] = vals

        g1.wait()
        g2.wait()
        o1 = pltpu.async_copy(
            ui_v, ui_out.at[pl.ds(base, _BPW), pl.ds(0, _D)], sem_c)
        o2 = pltpu.async_copy(
            vj_v, vj_out.at[pl.ds(base, _BPW), pl.ds(0, _D)], sem_c)

        gs, outs = [], []
        for kk in range(_K):
            slot = kk % 2
            if kk >= 2:
                outs[kk - 2].wait()
            gs.append(pltpu.async_copy(
                ei_hbm.at[kidx_v.at[pl.ds(kk * _BPW, _BPW)]],
                vk_v.at[slot], sem_b))
            if kk >= 1:
                gs[kk - 1].wait()
                pslot = (kk - 1) % 2
                outs.append(pltpu.async_copy(
                    vk_v.at[pslot],
                    vk_out.at[pl.ds((kk - 1) * _B + base, _BPW),
                              pl.ds(0, _D)], sem_c))
        gs[-1].wait()
        outs.append(pltpu.async_copy(
            vk_v.at[(_K - 1) % 2],
            vk_out.at[pl.ds((_K - 1) * _B + base, _BPW), pl.ds(0, _D)],
            sem_c))

        o1.wait()
        o2.wait()
        for o in outs[-2:]:
            o.wait()

    return sc_kern(user_idx, item_idx, negk, emb_user, emb_item)


def _clamp(e):
    n = jnp.sqrt(jnp.sum(e * e, axis=1, keepdims=True))
    return e * jnp.minimum(1.0, 1.0 / jnp.maximum(n, 1e-7))


_NB = 4
_BB = _B // _NB


def _fire_block(nb, usm_ref, tsm_ref, tab_ref, tiles_ref, sem):
    for i in range(_BB):
        u = usm_ref[nb * _BB + i]
        t = tsm_ref[nb * _BB + i]
        t0 = pl.multiple_of(t & -8, 8)
        u0 = pl.multiple_of((u // 128) * 128, 128)
        pltpu.make_async_copy(
            tab_ref.at[pl.ds(t0, 8), pl.ds(u0, 128)],
            tiles_ref.at[i], sem).start()


def _drain_block(tab_ref, tiles_ref, sem):
    for i in range(_BB):
        pltpu.make_async_copy(
            tab_ref.at[pl.ds(0, 8), pl.ds(0, 128)],
            tiles_ref.at[i], sem).wait()


def _dense_body(ui_ref, vj_ref, vk_ref, t_ref, u_ref, usm_ref, tsm_ref,
                tab_ref, x_ref, W1_ref, b1_ref, W2_ref, b2_ref, Wf_ref,
                bf_ref, out_ref, S_ref, m_ref, acc_ref, tiles0_ref, tiles1_ref,
                sem0, sem1):
    b = pl.program_id(0)

    @pl.when(b == 0)
    def _():
        S_ref[...] = jnp.zeros_like(S_ref)
        m_ref[...] = jnp.zeros_like(m_ref)
        acc_ref[0] = 0.0
        acc_ref[1] = 0.0
        _fire_block(b, usm_ref, tsm_ref, tab_ref, tiles0_ref, sem0)

    even = b % 2 == 0

    @pl.when(even & (b + 1 < _NB))
    def _():
        _fire_block(b + 1, usm_ref, tsm_ref, tab_ref, tiles1_ref, sem1)

    @pl.when((~even) & (b + 1 < _NB))
    def _():
        _fire_block(b + 1, usm_ref, tsm_ref, tab_ref, tiles0_ref, sem0)

    ui = _clamp(ui_ref[...][:, :_D])
    vj = _clamp(vj_ref[...][:, :_D])
    pos = jnp.sum((ui - vj) ** 2, axis=1, keepdims=True)
    hacc = jnp.zeros((_BB, 1), jnp.float32)
    for k in range(_K):
        vk = _clamp(vk_ref[k][:, :_D])
        negd = jnp.sum((ui - vk) ** 2, axis=1, keepdims=True)
        hacc += jnp.maximum(pos - negd + _MARGIN, 0.0)
    t = t_ref[...]
    u = u_ref[...]
    rem_t = (t & 7).reshape(_BB, 1, 1)
    rem_u = (u & 127).reshape(_BB, 1, 1)
    rr = lax.broadcasted_iota(jnp.int32, (_BB, 8, 128), 1)
    cc = lax.broadcasted_iota(jnp.int32, (_BB, 8, 128), 2)
    msk = (rr == rem_t) & (cc == rem_u)

    def _select(tiles_ref):
        sel = jnp.where(msk, tiles_ref[...], 0.0)
        return jnp.sum(jnp.sum(sel, axis=2), axis=1, keepdims=True)

    @pl.when(even)
    def _():
        _drain_block(tab_ref, tiles0_ref, sem0)

    @pl.when(~even)
    def _():
        _drain_block(tab_ref, tiles1_ref, sem1)

    w = jnp.where(even, _select(tiles0_ref), _select(tiles1_ref))
    sum_m_b = jnp.sum(w * hacc)

    h1 = jnp.maximum(
        jnp.dot(x_ref[...], W1_ref[...], preferred_element_type=jnp.float32)
        + b1_ref[...], 0.0)
    h2 = jnp.maximum(
        jnp.dot(h1, W2_ref[...], preferred_element_type=jnp.float32)
        + b2_ref[...], 0.0)
    feat = jnp.dot(h2, Wf_ref[...], preferred_element_type=jnp.float32) + bf_ref[...]
    sum_f_b = jnp.sum((feat - vj) ** 2)

    acc_ref[0] += sum_m_b
    acc_ref[1] += sum_f_b
    dn = (((0,), (0,)), ((), ()))
    S_ref[...] += (
        lax.dot_general(ui, ui, dn, preferred_element_type=jnp.float32)
        + lax.dot_general(vj, vj, dn, preferred_element_type=jnp.float32))
    m_ref[...] += (jnp.sum(ui, axis=0, keepdims=True)
                   + jnp.sum(vj, axis=0, keepdims=True))

    @pl.when(b == _NB - 1)
    def _():
        S = S_ref[...]
        mu = m_ref[...] / (2.0 * _B)
        C = (S - (2.0 * _B) * lax.dot_general(
            mu, mu, dn, preferred_element_type=jnp.float32)) / _B
        c2 = C * C
        norm_c = jnp.sqrt(jnp.sum(c2))
        ii = lax.broadcasted_iota(jnp.int32, (_D, _D), 0)
        jj = lax.broadcasted_iota(jnp.int32, (_D, _D), 1)
        norm_d = jnp.sqrt(jnp.sum(jnp.where(ii == jj, c2, 0.0)))
        loss_c = (norm_c - norm_d) / _B
        sm = acc_ref[0]
        sf = acc_ref[1]
        total = sm + sf * _LAMBDA_F + loss_c * _LAMBDA_C
        col = lax.broadcasted_iota(jnp.int32, (1, 128), 1)
        out_ref[...] = (jnp.where(col == 0, sm, 0.0)
                        + jnp.where(col == 1, sf, 0.0)
                        + jnp.where(col == 2, loss_c, 0.0)
                        + jnp.where(col == 3, total, 0.0))


def _tc_dense(ui_raw, vj_raw, vk3, t2, u2, user_idx, item_idx, tabT, item_x,
              W1, b1, W2, b2, Wf, bf):
    return pl.pallas_call(
        _dense_body,
        grid=(_NB,),
        in_specs=[
            pl.BlockSpec((_BB, 128), lambda b: (b, 0)),
            pl.BlockSpec((_BB, 128), lambda b: (b, 0)),
            pl.BlockSpec((_K, _BB, 128), lambda b: (0, b, 0)),
            pl.BlockSpec((_BB, 1), lambda b: (b, 0)),
            pl.BlockSpec((_BB, 1), lambda b: (b, 0)),
            pl.BlockSpec(memory_space=pltpu.SMEM),
            pl.BlockSpec(memory_space=pltpu.SMEM),
            pl.BlockSpec(memory_space=pl.ANY),
            pl.BlockSpec((_BB, 128), lambda b: (b, 0)),
            pl.BlockSpec((128, 256), lambda b: (0, 0)),
            pl.BlockSpec((1, 256), lambda b: (0, 0)),
            pl.BlockSpec((256, _D), lambda b: (0, 0)),
            pl.BlockSpec((1, _D), lambda b: (0, 0)),
            pl.BlockSpec((_D, 1), lambda b: (0, 0)),
            pl.BlockSpec((1, 1), lambda b: (0, 0)),
        ],
        out_specs=pl.BlockSpec((1, 128), lambda b: (0, 0)),
        out_shape=jax.ShapeDtypeStruct((1, 128), jnp.float32),
        scratch_shapes=[
            pltpu.VMEM((_D, _D), jnp.float32),
            pltpu.VMEM((1, _D), jnp.float32),
            pltpu.SMEM((2,), jnp.float32),
            pltpu.VMEM((_BB, 8, 128), jnp.float32),
            pltpu.VMEM((_BB, 8, 128), jnp.float32),
            pltpu.SemaphoreType.DMA,
            pltpu.SemaphoreType.DMA,
        ],
    )(ui_raw, vj_raw, vk3, t2, u2, user_idx, item_idx, tabT, item_x,
      W1, b1, W2, b2, Wf, bf)


def kernel(user_idx, item_idx, neg_item_idx, item_x, emb_user, emb_item,
           rank_d_ij, W1, b1, W2, b2, Wf, bf):
    negf = neg_item_idx.reshape(-1)
    ui_raw, vj_raw, vk_raw = _sc_gather(
        user_idx, item_idx, negf, emb_user, emb_item)
    vk3 = vk_raw.reshape(_K, _B, 128)
    out = _tc_dense(ui_raw, vj_raw, vk3, item_idx.reshape(_B, 1),
                    user_idx.reshape(_B, 1), user_idx, item_idx,
                    rank_d_ij.T, item_x,
                    W1, b1.reshape(1, -1), W2, b2.reshape(1, -1),
                    Wf, bf.reshape(1, 1))
    return (out[0, 0], out[0, 1], out[0, 2], out[0, 3])


# speedup vs baseline: 1.4540x; 1.4540x over previous
import functools

import jax
import jax.numpy as jnp
from jax import lax
from jax.experimental import pallas as pl
from jax.experimental.pallas import tpu as pltpu
from jax.experimental.pallas import tpu_sc as plsc

_B = 4096
_K = 20
_D = 64
_ITEM_N = 1000
_MARGIN = 0.5
_LAMBDA_F = 0.5
_LAMBDA_C = 1.0

_info = plsc.get_sparse_core_info()
_NC, _NS = _info.num_cores, _info.num_subcores
_NW = _NC * _NS
_BPW = _B // _NW


def _sc_gather(user_idx, item_idx, negk, emb_user, emb_item):
    mesh = plsc.VectorSubcoreMesh(core_axis_name="c", subcore_axis_name="s")

    @functools.partial(
        pl.kernel,
        out_type=[
            jax.ShapeDtypeStruct((_B, 128), jnp.float32),
            jax.ShapeDtypeStruct((_B, 128), jnp.float32),
            jax.ShapeDtypeStruct((_B * _K, 128), jnp.float32),
        ],
        mesh=mesh,
        compiler_params=pltpu.CompilerParams(
            use_tc_tiling_on_sc=False, needs_layout_passes=False),
        scratch_types=[
            pltpu.VMEM((_BPW,), jnp.int32),
            pltpu.VMEM((_BPW,), jnp.int32),
            pltpu.VMEM((_K * _BPW,), jnp.int32),
            pltpu.VMEM((_K * _BPW,), jnp.int32),
            pltpu.VMEM((_BPW, _D), jnp.float32),
            pltpu.VMEM((_BPW, _D), jnp.float32),
            pltpu.VMEM((2, _BPW, _D), jnp.float32),
            pltpu.SemaphoreType.DMA,
            pltpu.SemaphoreType.DMA,
            pltpu.SemaphoreType.DMA,
        ],
    )
    def sc_kern(uidx_hbm, iidx_hbm, nidx_hbm, eu_hbm, ei_hbm,
                ui_out, vj_out, vk_out,
                uidx_v, iidx_v, nraw_v, kidx_v, ui_v, vj_v, vk_v,
                sem_a, sem_b, sem_c):
        wid = lax.axis_index("s") * _NC + lax.axis_index("c")
        base = wid * _BPW

        cps = [
            pltpu.async_copy(uidx_hbm.at[pl.ds(base, _BPW)], uidx_v, sem_a),
            pltpu.async_copy(iidx_hbm.at[pl.ds(base, _BPW)], iidx_v, sem_a),
            pltpu.async_copy(
                nidx_hbm.at[pl.ds(base * _K, _BPW * _K)], nraw_v, sem_a),
        ]
        for c in cps:
            c.wait()

        g1 = pltpu.async_copy(eu_hbm.at[uidx_v], ui_v, sem_b)
        g2 = pltpu.async_copy(ei_hbm.at[iidx_v], vj_v, sem_b)

        lane = jnp.arange(16, dtype=jnp.int32) * _K
        for kk in range(_K):
            for c in range(_BPW // 16):
                vals = plsc.load_gather(nraw_v, [lane + (c * 16 * _K + kk)])
                kidx_v[pl.ds(kk * _BPW + c * 16, 16)] = vals

        g1.wait()
        g2.wait()
        o1 = pltpu.async_copy(
            ui_v, ui_out.at[pl.ds(base, _BPW), pl.ds(0, _D)], sem_c)
        o2 = pltpu.async_copy(
            vj_v, vj_out.at[pl.ds(base, _BPW), pl.ds(0, _D)], sem_c)

        gs, outs = [], []
        for kk in range(_K):
            slot = kk % 2
            if kk >= 2:
                outs[kk - 2].wait()
            gs.append(pltpu.async_copy(
                ei_hbm.at[kidx_v.at[pl.ds(kk * _BPW, _BPW)]],
                vk_v.at[slot], sem_b))
            if kk >= 1:
                gs[kk - 1].wait()
                pslot = (kk - 1) % 2
                outs.append(pltpu.async_copy(
                    vk_v.at[pslot],
                    vk_out.at[pl.ds((kk - 1) * _B + base, _BPW),
                              pl.ds(0, _D)], sem_c))
        gs[-1].wait()
        outs.append(pltpu.async_copy(
            vk_v.at[(_K - 1) % 2],
            vk_out.at[pl.ds((_K - 1) * _B + base, _BPW), pl.ds(0, _D)],
            sem_c))

        o1.wait()
        o2.wait()
        for o in outs[-2:]:
            o.wait()

    return sc_kern(user_idx, item_idx, negk, emb_user, emb_item)


def _clamp(e):
    n = jnp.sqrt(jnp.sum(e * e, axis=1, keepdims=True))
    return e * jnp.minimum(1.0, 1.0 / jnp.maximum(n, 1e-7))


_NB = 8
_BB = _B // _NB


def _fire_block(nb, usm_ref, tsm_ref, tab_ref, tiles_ref, sem):
    for i in range(_BB):
        u = usm_ref[nb * _BB + i]
        t = tsm_ref[nb * _BB + i]
        t0 = pl.multiple_of(t & -8, 8)
        u0 = pl.multiple_of((u // 128) * 128, 128)
        pltpu.make_async_copy(
            tab_ref.at[pl.ds(t0, 8), pl.ds(u0, 128)],
            tiles_ref.at[i], sem).start()


def _drain_block(tab_ref, tiles_ref, sem):
    for i in range(_BB):
        pltpu.make_async_copy(
            tab_ref.at[pl.ds(0, 8), pl.ds(0, 128)],
            tiles_ref.at[i], sem).wait()


def _dense_body(ui_ref, vj_ref, vk_ref, t_ref, u_ref, usm_ref, tsm_ref,
                tab_ref, x_ref, W1_ref, b1_ref, W2_ref, b2_ref, Wf_ref,
                bf_ref, out_ref, S_ref, m_ref, acc_ref, tiles0_ref, tiles1_ref,
                sem0, sem1):
    b = pl.program_id(0)

    @pl.when(b == 0)
    def _():
        S_ref[...] = jnp.zeros_like(S_ref)
        m_ref[...] = jnp.zeros_like(m_ref)
        acc_ref[0] = 0.0
        acc_ref[1] = 0.0
        _fire_block(b, usm_ref, tsm_ref, tab_ref, tiles0_ref, sem0)

    even = b % 2 == 0

    @pl.when(even & (b + 1 < _NB))
    def _():
        _fire_block(b + 1, usm_ref, tsm_ref, tab_ref, tiles1_ref, sem1)

    @pl.when((~even) & (b + 1 < _NB))
    def _():
        _fire_block(b + 1, usm_ref, tsm_ref, tab_ref, tiles0_ref, sem0)

    ui = _clamp(ui_ref[...][:, :_D])
    vj = _clamp(vj_ref[...][:, :_D])
    pos = jnp.sum((ui - vj) ** 2, axis=1, keepdims=True)
    hacc = jnp.zeros((_BB, 1), jnp.float32)
    for k in range(_K):
        vk = _clamp(vk_ref[k][:, :_D])
        negd = jnp.sum((ui - vk) ** 2, axis=1, keepdims=True)
        hacc += jnp.maximum(pos - negd + _MARGIN, 0.0)
    t = t_ref[...]
    u = u_ref[...]
    rem_t = (t & 7).reshape(_BB, 1, 1)
    rem_u = (u & 127).reshape(_BB, 1, 1)
    rr = lax.broadcasted_iota(jnp.int32, (_BB, 8, 128), 1)
    cc = lax.broadcasted_iota(jnp.int32, (_BB, 8, 128), 2)
    msk = (rr == rem_t) & (cc == rem_u)

    def _select(tiles_ref):
        sel = jnp.where(msk, tiles_ref[...], 0.0)
        return jnp.sum(jnp.sum(sel, axis=2), axis=1, keepdims=True)

    @pl.when(even)
    def _():
        _drain_block(tab_ref, tiles0_ref, sem0)

    @pl.when(~even)
    def _():
        _drain_block(tab_ref, tiles1_ref, sem1)

    w = jnp.where(even, _select(tiles0_ref), _select(tiles1_ref))
    sum_m_b = jnp.sum(w * hacc)

    h1 = jnp.maximum(
        jnp.dot(x_ref[...], W1_ref[...], preferred_element_type=jnp.float32)
        + b1_ref[...], 0.0)
    h2 = jnp.maximum(
        jnp.dot(h1, W2_ref[...], preferred_element_type=jnp.float32)
        + b2_ref[...], 0.0)
    feat = jnp.dot(h2, Wf_ref[...], preferred_element_type=jnp.float32) + bf_ref[...]
    sum_f_b = jnp.sum((feat - vj) ** 2)

    acc_ref[0] += sum_m_b
    acc_ref[1] += sum_f_b
    dn = (((0,), (0,)), ((), ()))
    S_ref[...] += (
        lax.dot_general(ui, ui, dn, preferred_element_type=jnp.float32)
        + lax.dot_general(vj, vj, dn, preferred_element_type=jnp.float32))
    m_ref[...] += (jnp.sum(ui, axis=0, keepdims=True)
                   + jnp.sum(vj, axis=0, keepdims=True))

    @pl.when(b == _NB - 1)
    def _():
        S = S_ref[...]
        mu = m_ref[...] / (2.0 * _B)
        C = (S - (2.0 * _B) * lax.dot_general(
            mu, mu, dn, preferred_element_type=jnp.float32)) / _B
        c2 = C * C
        norm_c = jnp.sqrt(jnp.sum(c2))
        ii = lax.broadcasted_iota(jnp.int32, (_D, _D), 0)
        jj = lax.broadcasted_iota(jnp.int32, (_D, _D), 1)
        norm_d = jnp.sqrt(jnp.sum(jnp.where(ii == jj, c2, 0.0)))
        loss_c = (norm_c - norm_d) / _B
        sm = acc_ref[0]
        sf = acc_ref[1]
        total = sm + sf * _LAMBDA_F + loss_c * _LAMBDA_C
        col = lax.broadcasted_iota(jnp.int32, (1, 128), 1)
        out_ref[...] = (jnp.where(col == 0, sm, 0.0)
                        + jnp.where(col == 1, sf, 0.0)
                        + jnp.where(col == 2, loss_c, 0.0)
                        + jnp.where(col == 3, total, 0.0))


def _tc_dense(ui_raw, vj_raw, vk3, t2, u2, user_idx, item_idx, tabT, item_x,
              W1, b1, W2, b2, Wf, bf):
    return pl.pallas_call(
        _dense_body,
        grid=(_NB,),
        in_specs=[
            pl.BlockSpec((_BB, 128), lambda b: (b, 0)),
            pl.BlockSpec((_BB, 128), lambda b: (b, 0)),
            pl.BlockSpec((_K, _BB, 128), lambda b: (0, b, 0)),
            pl.BlockSpec((_BB, 1), lambda b: (b, 0)),
            pl.BlockSpec((_BB, 1), lambda b: (b, 0)),
            pl.BlockSpec(memory_space=pltpu.SMEM),
            pl.BlockSpec(memory_space=pltpu.SMEM),
            pl.BlockSpec(memory_space=pl.ANY),
            pl.BlockSpec((_BB, 128), lambda b: (b, 0)),
            pl.BlockSpec((128, 256), lambda b: (0, 0)),
            pl.BlockSpec((1, 256), lambda b: (0, 0)),
            pl.BlockSpec((256, _D), lambda b: (0, 0)),
            pl.BlockSpec((1, _D), lambda b: (0, 0)),
            pl.BlockSpec((_D, 1), lambda b: (0, 0)),
            pl.BlockSpec((1, 1), lambda b: (0, 0)),
        ],
        out_specs=pl.BlockSpec((1, 128), lambda b: (0, 0)),
        out_shape=jax.ShapeDtypeStruct((1, 128), jnp.float32),
        scratch_shapes=[
            pltpu.VMEM((_D, _D), jnp.float32),
            pltpu.VMEM((1, _D), jnp.float32),
            pltpu.SMEM((2,), jnp.float32),
            pltpu.VMEM((_BB, 8, 128), jnp.float32),
            pltpu.VMEM((_BB, 8, 128), jnp.float32),
            pltpu.SemaphoreType.DMA,
            pltpu.SemaphoreType.DMA,
        ],
    )(ui_raw, vj_raw, vk3, t2, u2, user_idx, item_idx, tabT, item_x,
      W1, b1, W2, b2, Wf, bf)


def kernel(user_idx, item_idx, neg_item_idx, item_x, emb_user, emb_item,
           rank_d_ij, W1, b1, W2, b2, Wf, bf):
    negf = neg_item_idx.reshape(-1)
    ui_raw, vj_raw, vk_raw = _sc_gather(
        user_idx, item_idx, negf, emb_user, emb_item)
    vk3 = vk_raw.reshape(_K, _B, 128)
    out = _tc_dense(ui_raw, vj_raw, vk3, item_idx.reshape(_B, 1),
                    user_idx.reshape(_B, 1), user_idx, item_idx,
                    rank_d_ij.T, item_x,
                    W1, b1.reshape(1, -1), W2, b2.reshape(1, -1),
                    Wf, bf.reshape(1, 1))
    return (out[0, 0], out[0, 1], out[0, 2], out[0, 3])
